# Initial kernel scaffold; baseline (speedup 1.0000x reference)
import jax
import jax.numpy as jnp
from jax.experimental import pallas as pl


def kernel(toks, main_weight, special_weight):
    raise NotImplementedError("write your pallas kernel here")


# speedup vs baseline: 2.1145x; 2.1145x over previous
import functools
import jax
import jax.numpy as jnp
from jax import lax
from jax.experimental import pallas as pl
from jax.experimental.pallas import tpu as pltpu, tpu_sc as plsc

_CODES = 1000000
_SPECIAL = 1024
_WIDTH = 64

_NC = 2
_NS = 16
_NW = _NC * _NS
_LANES = 16

_B = 4096 * 50
_BPW = _B // _NW
_CHUNK = 320
_NCHUNK = _BPW // _CHUNK
_GATHER = 80
_NGATHER = _CHUNK // _GATHER
_NVREG = _CHUNK // _LANES
_NBUF = 3
_NVISIT = -(-_NCHUNK // _NBUF)


def _body(toks_hbm, main_hbm, spec_hbm, out_hbm,
          t0, t1, t2, m0, m1, m2, r0, r1, r2, spec_v,
          g0, g1, g2, o0, o1, o2):
    toks_v = (t0, t1, t2)
    midx_v = (m0, m1, m2)
    rows_v = (r0, r1, r2)
    gsem = (g0, g1, g2)
    osem = (o0, o1, o2)

    wid = lax.axis_index("s") * _NC + lax.axis_index("c")
    base = wid * _BPW

    pltpu.sync_copy(spec_hbm, spec_v)

    def stage_and_fire(ch, b):
        start = base + ch * _CHUNK
        pltpu.sync_copy(toks_hbm.at[pl.ds(start, _CHUNK)], toks_v[b])

        def idx_body(i, _):
            t = toks_v[b][pl.ds(i * _LANES, _LANES)]
            midx_v[b][pl.ds(i * _LANES, _LANES)] = jnp.where(t >= _CODES, 0, t)
            return 0

        lax.fori_loop(0, _NVREG, idx_body, 0)
        for j in range(_NGATHER):
            pltpu.async_copy(
                main_hbm.at[midx_v[b].at[pl.ds(j * _GATHER, _GATHER)]],
                rows_v[b].at[pl.ds(j * _GATHER, _GATHER)], gsem[b])

    def drain_gathers(b):
        for j in range(_NGATHER):
            pltpu.make_async_copy(
                main_hbm.at[midx_v[b].at[pl.ds(j * _GATHER, _GATHER)]],
                rows_v[b].at[pl.ds(j * _GATHER, _GATHER)], gsem[b]).wait()

    def patch(b):
        def spec_body(i, _):
            t = toks_v[b][pl.ds(i * _LANES, _LANES)]
            m = t >= _CODES
            cnt = plsc.all_reduce_population_count(m)

            @pl.when(cnt[0] > 0)
            def _():
                sidx = jnp.where(m, t - _CODES, 0)
                pos = i * _LANES + lax.iota(jnp.int32, _LANES)
                for c in range(_WIDTH):
                    col = jnp.full((_LANES,), c, jnp.int32)
                    vals = plsc.load_gather(spec_v, [sidx, col])
                    plsc.store_scatter(rows_v[b], [pos, col], vals, mask=m)

            return 0

        lax.fori_loop(0, _NVREG, spec_body, 0)

    def fire_out(ch, b):
        start = base + ch * _CHUNK
        pltpu.async_copy(rows_v[b], out_hbm.at[pl.ds(start, _CHUNK)], osem[b])

    def wait_out(ch, b):
        start = base + ch * _CHUNK
        pltpu.make_async_copy(
            rows_v[b], out_hbm.at[pl.ds(start, _CHUNK)], osem[b]).wait()

    stage_and_fire(0, 0)
    stage_and_fire(1, 1)

    def ring_body(g, _):
        for b in range(_NBUF):
            ch = g * _NBUF + b

            @pl.when(ch < _NCHUNK)
            def _():
                drain_gathers(b)
                patch(b)
                fire_out(ch, b)
                nxt = ch + 2
                b2 = (b + 2) % _NBUF

                @pl.when(nxt < _NCHUNK)
                def _():
                    @pl.when(nxt >= _NBUF)
                    def _():
                        wait_out(nxt - _NBUF, b2)

                    stage_and_fire(nxt, b2)

        return 0

    lax.fori_loop(0, _NVISIT, ring_body, 0)

    for k in range(_NCHUNK - _NBUF, _NCHUNK):
        wait_out(k, k % _NBUF)


_sc_call = functools.partial(
    pl.kernel,
    out_type=jax.ShapeDtypeStruct((_B, _WIDTH), jnp.float32),
    mesh=plsc.VectorSubcoreMesh(core_axis_name="c", subcore_axis_name="s"),
    compiler_params=pltpu.CompilerParams(
        needs_layout_passes=False, use_tc_tiling_on_sc=False),
    scratch_types=(
        [pltpu.VMEM((_CHUNK,), jnp.int32)] * 3
        + [pltpu.VMEM((_CHUNK,), jnp.int32)] * 3
        + [pltpu.VMEM((_CHUNK, _WIDTH), jnp.float32)] * 3
        + [pltpu.VMEM((_SPECIAL, _WIDTH), jnp.float32)]
        + [pltpu.SemaphoreType.DMA] * 6
    ),
)(_body)


@jax.jit
def kernel(toks, main_weight, special_weight):
    embs = _sc_call(toks.reshape(-1), main_weight, special_weight)
    return embs.reshape(toks.shape + (_WIDTH,))


# speedup vs baseline: 2.5346x; 1.1987x over previous
import functools
import jax
import jax.numpy as jnp
from jax import lax
from jax.experimental import pallas as pl
from jax.experimental.pallas import tpu as pltpu, tpu_sc as plsc

_CODES = 1000000
_SPECIAL = 1024
_WIDTH = 64

_NC = 2
_NS = 16
_NW = _NC * _NS
_L = 16

_NB = 4096
_NH = 50
_B = _NB * _NH
_BPW = _B // _NW

_BKT_SHIFT = 15
_PIECE = 512
_PPB = 64
_REG = 6912
_CAPA = 10752
_CAPP = _CAPA + 1024
_ROWS = 64

_I32 = jnp.int32


def _lane_iota():
    return lax.iota(_I32, _L)


def _sort_runs(tmp, key):
    li = _lane_iota()
    sk, sl = plsc.sort_key_val(key, li)
    tmp[...] = sk
    prev = plsc.load_gather(tmp, [jnp.maximum(li - 1, 0)])
    nxt = plsc.load_gather(tmp, [jnp.minimum(li + 1, _L - 1)])
    isstart = (li == 0) | (sk != prev)
    islast = (li == _L - 1) | (sk != nxt)
    startpos = plsc.cummax(jnp.where(isstart, li, 0))
    rank = li - startpos
    return sk, sl, rank, islast


def _permute(tmp, x, sl):
    tmp[...] = x
    return plsc.load_gather(tmp, [sl])


def _zero(ref, n):
    z = jnp.zeros((_L,), _I32)
    for i in range(n // _L):
        ref[pl.ds(i * _L, _L)] = z


def _sload(ref, idx):
    v = plsc.load_gather(ref, [jnp.full((_L,), 0, _I32) + idx])
    return v[0]


def _excl_prefix_align8(src_ref, dst_ref, n):
    carry = jnp.zeros((), _I32)
    for i in range(n // _L):
        a = src_ref[pl.ds(i * _L, _L)]
        a8 = (a + 7) & ~7
        c = plsc.cumsum(a8)
        dst_ref[pl.ds(i * _L, _L)] = c - a8 + carry
        carry = carry + c[_L - 1]


def _k1_body(tT_hbm, spos_hbm, stok_hbm, cnts_hbm, offs_hbm,
             tTv, spos_v, stok_v, hist, offs, runoffs, tmp):
    wid = lax.axis_index("s") * _NC + lax.axis_index("c")
    b0 = wid * (_NB // _NW)

    pltpu.sync_copy(tT_hbm.at[:, pl.ds(b0, _NB // _NW)], tTv)

    _zero(hist, 32)

    def tok_pos(idx):
        h = idx // 8
        iv = idx - h * 8
        t = tTv[h, pl.ds(iv * _L, _L)]
        pos = (b0 + iv * _L + _lane_iota()) * _NH + h
        return t, pos

    def pass1(idx, _):
        t, _pos = tok_pos(idx)
        d = jnp.where(t >= _CODES, 31, lax.shift_right_logical(t, _BKT_SHIFT))
        sd, _sl, rank, islast = _sort_runs(tmp, d)
        plsc.addupdate_scatter(hist, [sd], rank + 1, mask=islast)
        return 0

    lax.fori_loop(0, _NH * 8, pass1, 0)

    _excl_prefix_align8(hist, offs, 32)
    for i in range(2):
        runoffs[pl.ds(i * _L, _L)] = offs[pl.ds(i * _L, _L)]

    def pass2(idx, _):
        t, pos = tok_pos(idx)
        d = jnp.where(t >= _CODES, 31, lax.shift_right_logical(t, _BKT_SHIFT))
        sd, sl, rank, islast = _sort_runs(tmp, d)
        t_s = _permute(tmp, t, sl)
        pos_s = _permute(tmp, pos, sl)
        base = plsc.load_gather(runoffs, [sd])
        slot = base + rank
        plsc.store_scatter(spos_v, [slot], pos_s)
        plsc.store_scatter(stok_v, [slot], t_s)
        plsc.addupdate_scatter(runoffs, [sd], rank + 1, mask=islast)
        return 0

    lax.fori_loop(0, _NH * 8, pass2, 0)

    pltpu.sync_copy(spos_v, spos_hbm.at[pl.ds(wid * _REG, _REG)])
    pltpu.sync_copy(stok_v, stok_hbm.at[pl.ds(wid * _REG, _REG)])
    pltpu.sync_copy(hist, cnts_hbm.at[pl.ds(wid * 32, 32)])
    pltpu.sync_copy(offs, offs_hbm.at[pl.ds(wid * 32, 32)])


def _k2_body(mwT_hbm, swT_hbm, edgeT_hbm, spos_hbm, stok_hbm, cnts_hbm,
             offs_hbm, out_hbm,
             cpos, ctok, ppos, ptok, pv0, pv1, st0, st1, pb0, pb1,
             hist2, offs2, runoffs2, cnts_all, offs_all, tmp,
             ps0, ps1, ssem):
    pvs = (pv0, pv1)
    sts = (st0, st1)
    pbs = (pb0, pb1)
    pss = (ps0, ps1)
    d = lax.axis_index("s") * _NC + lax.axis_index("c")

    pltpu.sync_copy(cnts_hbm, cnts_all)
    pltpu.sync_copy(offs_hbm, offs_all)
    cursor = jnp.zeros((), _I32)
    for w in range(_NW):
        n = _sload(cnts_all, w * 32 + d)
        o = pl.multiple_of(_sload(offs_all, w * 32 + d), 8)
        nch = lax.shift_right_logical(n + 511, 9)
        cur = cursor

        def copy_chunk(ch, _, w=w, o=o, cur=cur):
            src = pl.multiple_of(w * _REG + o + ch * 512, 8)
            dst = pl.multiple_of(jnp.minimum(cur + ch * 512, _CAPA - 512), 8)
            pltpu.sync_copy(spos_hbm.at[pl.ds(src, 512)],
                            cpos.at[pl.ds(dst, 512)])
            pltpu.sync_copy(stok_hbm.at[pl.ds(src, 512)],
                            ctok.at[pl.ds(dst, 512)])
            return 0

        lax.fori_loop(0, nch, copy_chunk, 0)

        end = cursor + n
        aend = (end + 7) & ~7

        @pl.when(n > 0)
        def _():
            p0 = _sload(cpos, jnp.minimum(cursor, _CAPA - 1))
            t0 = _sload(ctok, jnp.minimum(cursor, _CAPA - 1))
            idxv = end + _lane_iota()
            m = idxv < aend
            idxc = jnp.minimum(idxv, _CAPA - 1)
            plsc.store_scatter(cpos, [idxc], jnp.full((_L,), 0, _I32) + p0,
                               mask=m)
            plsc.store_scatter(ctok, [idxc], jnp.full((_L,), 0, _I32) + t0,
                               mask=m)

        cursor = pl.multiple_of(jnp.minimum(aend, _CAPA - 512), 8)

    n_d = cursor

    _zero(hist2, 80)

    def lp_of(t):
        return jnp.where(t >= _CODES,
                         lax.shift_right_logical(t - _CODES, 9),
                         lax.shift_right_logical(t, 9) & 63)

    nvr = lax.shift_right_logical(n_d + _L - 1, 4)

    def passB(v, _):
        g = v * _L
        t = ctok[pl.ds(g, _L)]
        valid = (g + _lane_iota()) < n_d
        lp = jnp.where(valid, lp_of(t), 64)
        sd, _sl, rank, islast = _sort_runs(tmp, lp)
        plsc.addupdate_scatter(hist2, [sd], rank + 1,
                               mask=islast & (sd < 64))
        return 0

    lax.fori_loop(0, nvr, passB, 0)

    _excl_prefix_align8(hist2, offs2, 80)
    for i in range(5):
        runoffs2[pl.ds(i * _L, _L)] = offs2[pl.ds(i * _L, _L)]

    def passC(v, _):
        g = v * _L
        t = ctok[pl.ds(g, _L)]
        pos = cpos[pl.ds(g, _L)]
        valid = (g + _lane_iota()) < n_d
        lp = jnp.where(valid, lp_of(t), 64)
        sd, sl, rank, islast = _sort_runs(tmp, lp)
        t_s = _permute(tmp, t, sl)
        pos_s = _permute(tmp, pos, sl)
        v_s = sd < 64
        base = plsc.load_gather(runoffs2, [sd])
        slot = jnp.minimum(base + rank, _CAPP - 1)
        plsc.store_scatter(ppos, [slot], pos_s, mask=v_s)
        plsc.store_scatter(ptok, [slot], t_s, mask=v_s)
        plsc.addupdate_scatter(runoffs2, [sd], rank + 1, mask=islast & v_s)
        return 0

    lax.fori_loop(0, nvr, passC, 0)

    is_spec = d == 31

    def fire_piece(p, b):
        edge = (~is_spec) & (d == 30) & (p == 33)
        main_off = pl.multiple_of(
            jnp.minimum(d * 32768 + p * _PIECE, _CODES - 64 - _PIECE), 128)
        spec_off = pl.multiple_of(jnp.minimum(p, 1) * _PIECE, 128)

        @pl.when(is_spec)
        def _():
            pltpu.async_copy(swT_hbm.at[:, pl.ds(spec_off, _PIECE)],
                             pvs[b], pss[b])

        @pl.when(edge)
        def _():
            pltpu.async_copy(edgeT_hbm, pvs[b], pss[b])

        @pl.when((~is_spec) & (~edge))
        def _():
            pltpu.async_copy(mwT_hbm.at[:, pl.ds(main_off, _PIECE)],
                             pvs[b], pss[b])

    def wait_piece(b):
        pltpu.make_async_copy(mwT_hbm.at[:, pl.ds(0, _PIECE)],
                              pvs[b], pss[b]).wait()

    def wait_scatter_one():
        pltpu.make_async_copy(sts[0], out_hbm.at[pbs[0]], ssem).wait()

    fire_piece(0, 0)
    fire_piece(1, 1)

    def gbody(g, nf):
        for b in range(2):
            p = g * 2 + b
            wait_piece(b)
            scount = _sload(hist2, p)
            soff = pl.multiple_of(_sload(offs2, p), 8)
            lo = jnp.where(is_spec, _CODES, d * 32768) + p * _PIECE
            nck = lax.shift_right_logical(scount + _ROWS - 1, 6)
            npair = lax.shift_right_logical(nck + 1, 1)

            def cpair(cg, nf, b=b, scount=scount, soff=soff, lo=lo, nck=nck):
                for sb in range(2):
                    ch = cg * 2 + sb
                    fires = ch < nck

                    @pl.when(fires)
                    def _(ch=ch, sb=sb, nf=nf):
                        cbase = pl.multiple_of(soff + ch * _ROWS, 8)
                        v = scount - ch * _ROWS
                        p0 = _sload(ppos, cbase)
                        t0 = _sload(ptok, cbase)

                        for v8 in range(_ROWS // _L):
                            li = v8 * _L + _lane_iota()
                            m = li < v
                            t = ptok[pl.ds(cbase + v8 * _L, _L)]
                            po = ppos[pl.ds(cbase + v8 * _L, _L)]
                            t_e = jnp.where(m, t, t0)
                            p_e = jnp.where(m, po, p0)
                            pbs[sb][pl.ds(v8 * _L, _L)] = p_e
                            j = t_e - lo

                            def cbody(ci, _, j=j, li=li, b=b, sb=sb):
                                for cc in range(_L):
                                    col = jnp.full((_L,), cc, _I32) + ci * _L
                                    vals = plsc.load_gather(pvs[b], [col, j])
                                    plsc.store_scatter(sts[sb], [li, col],
                                                       vals)
                                return 0

                            lax.fori_loop(0, _WIDTH // _L, cbody, 0)

                        @pl.when(nf >= 1)
                        def _():
                            wait_scatter_one()

                        pltpu.async_copy(sts[sb], out_hbm.at[pbs[sb]], ssem)

                    nf = nf + jnp.where(fires, 1, 0)
                return nf

            nf = lax.fori_loop(0, npair, cpair, nf)

            @pl.when(p + 2 < _PPB)
            def _(p=p, b=b):
                fire_piece(p + 2, b)

        return nf

    nf = lax.fori_loop(0, _PPB // 2, gbody, jnp.zeros((), _I32))

    @pl.when(nf >= 1)
    def _():
        wait_scatter_one()


_params = pltpu.CompilerParams(
    needs_layout_passes=False, use_tc_tiling_on_sc=True)
_mesh = plsc.VectorSubcoreMesh(core_axis_name="c", subcore_axis_name="s")

_k1 = functools.partial(
    pl.kernel,
    out_type=(
        jax.ShapeDtypeStruct((_NW * _REG + 512,), _I32),
        jax.ShapeDtypeStruct((_NW * _REG + 512,), _I32),
        jax.ShapeDtypeStruct((_NW * 32,), _I32),
        jax.ShapeDtypeStruct((_NW * 32,), _I32),
    ),
    mesh=_mesh,
    compiler_params=_params,
    scratch_types=[
        pltpu.VMEM((_NH, _NB // _NW), _I32),
        pltpu.VMEM((_REG,), _I32),
        pltpu.VMEM((_REG,), _I32),
        pltpu.VMEM((32,), _I32),
        pltpu.VMEM((32,), _I32),
        pltpu.VMEM((32,), _I32),
        pltpu.VMEM((_L,), _I32),
    ],
)(_k1_body)

_k2 = functools.partial(
    pl.kernel,
    out_type=jax.ShapeDtypeStruct((_B, 128), jnp.float32),
    mesh=_mesh,
    compiler_params=_params,
    scratch_types=[
        pltpu.VMEM((_CAPA,), _I32),
        pltpu.VMEM((_CAPA,), _I32),
        pltpu.VMEM((_CAPP,), _I32),
        pltpu.VMEM((_CAPP,), _I32),
        pltpu.VMEM((_WIDTH, _PIECE), jnp.float32),
        pltpu.VMEM((_WIDTH, _PIECE), jnp.float32),
        pltpu.VMEM((_ROWS, 128), jnp.float32),
        pltpu.VMEM((_ROWS, 128), jnp.float32),
        pltpu.VMEM((_ROWS,), _I32),
        pltpu.VMEM((_ROWS,), _I32),
        pltpu.VMEM((80,), _I32),
        pltpu.VMEM((80,), _I32),
        pltpu.VMEM((80,), _I32),
        pltpu.VMEM((_NW * 32,), _I32),
        pltpu.VMEM((_NW * 32,), _I32),
        pltpu.VMEM((_L,), _I32),
        pltpu.SemaphoreType.DMA,
        pltpu.SemaphoreType.DMA,
        pltpu.SemaphoreType.DMA,
    ],
)(_k2_body)


@jax.jit
def kernel(toks, main_weight, special_weight):
    spos, stok, cnts, offs = _k1(toks.T)
    edgeT = jnp.pad(main_weight[_CODES - 64:].T, ((0, 0), (0, _PIECE - 64)))
    out128 = _k2(main_weight.T, special_weight.T, edgeT,
                 spos, stok, cnts, offs)
    return out128[:, :_WIDTH].reshape(_NB, _NH, _WIDTH)
